# Initial kernel scaffold; baseline (speedup 1.0000x reference)
import jax
import jax.numpy as jnp
from jax.experimental import pallas as pl


def kernel(x, edge_index, W1, b1, W2, b2):
    raise NotImplementedError("write your pallas kernel here")


# speedup vs baseline: 31.5720x; 31.5720x over previous
import functools
import jax
import jax.numpy as jnp
from jax import lax
from jax.experimental import pallas as pl
from jax.experimental.pallas import tpu as pltpu
from jax.experimental.pallas import tpu_sc as plsc

N = 10000
D = 128
H = 32
E = 320000
NP = 10240
NW = 32
CK = 128
CH = 80
EP = NW * CH * CK
RPT = NP // 16
NBUF = 4

_mesh = plsc.VectorSubcoreMesh(core_axis_name="c", subcore_axis_name="s")
_sc_params = pltpu.CompilerParams(use_tc_tiling_on_sc=False)


def _z16():
    return jnp.zeros((16,), jnp.float32)


def _worker_id():
    return lax.axis_index("s") * 2 + lax.axis_index("c")


@functools.partial(
    pl.kernel,
    out_type=jax.ShapeDtypeStruct((2, NP), jnp.float32),
    mesh=_mesh,
    compiler_params=_sc_params,
    scratch_types=[
        pltpu.VMEM((CH, CK), jnp.int32),
        pltpu.VMEM((CK,), jnp.float32),
        pltpu.VMEM((RPT,), jnp.float32),
        pltpu.VMEM_SHARED((NP,), jnp.float32),
    ],
)
def _deg_kernel(dst_hbm, out_hbm, dst_v, ones_v, zer_v, accum):
    cid = lax.axis_index("c")
    sid = lax.axis_index("s")
    wid = _worker_id()

    def fill_ones(i, _):
        ones_v[pl.ds(i * 16, 16)] = _z16() + 1.0
        return 0

    lax.fori_loop(0, CK // 16, fill_ones, 0)

    def fill_zero(i, _):
        zer_v[pl.ds(i * 16, 16)] = _z16()
        return 0

    lax.fori_loop(0, RPT // 16, fill_zero, 0)

    pltpu.sync_copy(zer_v, accum.at[pl.ds(sid * RPT, RPT)])
    pltpu.sync_copy(dst_hbm.at[wid], dst_v)
    plsc.subcore_barrier()

    def body(j, _):
        pltpu.sync_copy(ones_v, accum.at[dst_v.at[j]], add=True)
        return 0

    lax.fori_loop(0, CH, body, 0)
    plsc.subcore_barrier()
    pltpu.sync_copy(
        accum.at[pl.ds(sid * RPT, RPT)],
        out_hbm.at[cid, pl.ds(sid * RPT, RPT)],
    )


@functools.partial(
    pl.kernel,
    out_type=jax.ShapeDtypeStruct((2, NP, H), jnp.float32),
    mesh=_mesh,
    compiler_params=_sc_params,
    scratch_types=[
        pltpu.VMEM((CH, CK), jnp.int32),
        pltpu.VMEM((CH, CK), jnp.int32),
        pltpu.VMEM((NBUF, CK, H), jnp.float32),
        pltpu.VMEM_SHARED((NP, H), jnp.float32),
        pltpu.SemaphoreType.DMA,
        pltpu.SemaphoreType.DMA,
    ],
)
def _agg_feat_kernel(tbl_hbm, src_hbm, dst_hbm, out_hbm, src_v, dst_v, rows_v,
                     accum, gsem, ssem):
    cid = lax.axis_index("c")
    sid = lax.axis_index("s")
    wid = _worker_id()

    def fill_zero(i, _):
        rows_v[0, i, pl.ds(0, 16)] = _z16()
        rows_v[0, i, pl.ds(16, 16)] = _z16()
        return 0

    lax.fori_loop(0, CK, fill_zero, 0)

    def zcopy(i, _):
        pltpu.sync_copy(
            rows_v.at[0], accum.at[pl.ds(sid * RPT + i * CK, CK)]
        )
        return 0

    lax.fori_loop(0, RPT // CK, zcopy, 0)
    pltpu.sync_copy(src_hbm.at[wid], src_v)
    pltpu.sync_copy(dst_hbm.at[wid], dst_v)
    plsc.subcore_barrier()

    def group(g, _):
        base = g * NBUF
        gd = [
            pltpu.async_copy(
                tbl_hbm.at[src_v.at[base + b]], rows_v.at[b], gsem
            )
            for b in range(NBUF)
        ]
        for d in gd:
            d.wait()
        sd = [
            pltpu.async_copy(
                rows_v.at[b], accum.at[dst_v.at[base + b]], ssem, add=True
            )
            for b in range(NBUF)
        ]
        for d in sd:
            d.wait()
        return 0

    lax.fori_loop(0, CH // NBUF, group, 0)
    plsc.subcore_barrier()
    pltpu.sync_copy(
        accum.at[pl.ds(sid * RPT, RPT)],
        out_hbm.at[cid, pl.ds(sid * RPT, RPT)],
    )


@functools.partial(
    pl.kernel,
    out_type=jax.ShapeDtypeStruct((2, NP), jnp.float32),
    mesh=_mesh,
    compiler_params=_sc_params,
    scratch_types=[
        pltpu.VMEM((CH, CK), jnp.int32),
        pltpu.VMEM((CH, CK), jnp.int32),
        pltpu.VMEM((NBUF, CK), jnp.float32),
        pltpu.VMEM((RPT,), jnp.float32),
        pltpu.VMEM_SHARED((NP,), jnp.float32),
        pltpu.SemaphoreType.DMA,
        pltpu.SemaphoreType.DMA,
    ],
)
def _agg_scal_kernel(tbl_hbm, src_hbm, dst_hbm, out_hbm, src_v, dst_v, rows_v,
                     zer_v, accum, gsem, ssem):
    cid = lax.axis_index("c")
    sid = lax.axis_index("s")
    wid = _worker_id()

    def fill_zero(i, _):
        zer_v[pl.ds(i * 16, 16)] = _z16()
        return 0

    lax.fori_loop(0, RPT // 16, fill_zero, 0)

    pltpu.sync_copy(zer_v, accum.at[pl.ds(sid * RPT, RPT)])
    pltpu.sync_copy(src_hbm.at[wid], src_v)
    pltpu.sync_copy(dst_hbm.at[wid], dst_v)
    plsc.subcore_barrier()

    def group(g, _):
        base = g * NBUF
        gd = [
            pltpu.async_copy(
                tbl_hbm.at[src_v.at[base + b]], rows_v.at[b], gsem
            )
            for b in range(NBUF)
        ]
        for d in gd:
            d.wait()
        sd = [
            pltpu.async_copy(
                rows_v.at[b], accum.at[dst_v.at[base + b]], ssem, add=True
            )
            for b in range(NBUF)
        ]
        for d in sd:
            d.wait()
        return 0

    lax.fori_loop(0, CH // NBUF, group, 0)
    plsc.subcore_barrier()
    pltpu.sync_copy(
        accum.at[pl.ds(sid * RPT, RPT)],
        out_hbm.at[cid, pl.ds(sid * RPT, RPT)],
    )


BN = 1024


def _k1_body(x_ref, w1_ref, degt_ref, h1_ref, h1p_ref, dinv_ref):
    h1 = jnp.dot(x_ref[...], w1_ref[...], preferred_element_type=jnp.float32)
    degt = degt_ref[...]
    deg = degt[:, 0:1] + degt[:, 1:2] + 1.0
    dinv = lax.rsqrt(deg)
    h1_ref[...] = h1
    h1p_ref[...] = h1 * dinv
    dinv_ref[...] = dinv


def _k1(xpad, W1, degt):
    return pl.pallas_call(
        _k1_body,
        grid=(NP // BN,),
        in_specs=[
            pl.BlockSpec((BN, D), lambda i: (i, 0)),
            pl.BlockSpec((D, H), lambda i: (0, 0)),
            pl.BlockSpec((BN, 2), lambda i: (i, 0)),
        ],
        out_specs=[
            pl.BlockSpec((BN, H), lambda i: (i, 0)),
            pl.BlockSpec((BN, H), lambda i: (i, 0)),
            pl.BlockSpec((BN, 1), lambda i: (i, 0)),
        ],
        out_shape=[
            jax.ShapeDtypeStruct((NP, H), jnp.float32),
            jax.ShapeDtypeStruct((NP, H), jnp.float32),
            jax.ShapeDtypeStruct((NP, 1), jnp.float32),
        ],
    )(xpad, W1, degt)


def _k2_body(gp_ref, h1_ref, dinv_ref, b1_ref, w2t_ref, h_ref, z_ref):
    g = gp_ref[0] + gp_ref[1]
    dinv = dinv_ref[...]
    acc = dinv * (g + dinv * h1_ref[...]) + b1_ref[...]
    h = jnp.maximum(acc, 0.0)
    h_ref[...] = h
    h2 = jnp.sum(h * w2t_ref[...], axis=1, keepdims=True)
    z_ref[...] = dinv * h2


def _k2(gp, h1, dinv, b1r, w2t):
    return pl.pallas_call(
        _k2_body,
        grid=(NP // BN,),
        in_specs=[
            pl.BlockSpec((2, BN, H), lambda i: (0, i, 0)),
            pl.BlockSpec((BN, H), lambda i: (i, 0)),
            pl.BlockSpec((BN, 1), lambda i: (i, 0)),
            pl.BlockSpec((1, H), lambda i: (0, 0)),
            pl.BlockSpec((1, H), lambda i: (0, 0)),
        ],
        out_specs=[
            pl.BlockSpec((BN, H), lambda i: (i, 0)),
            pl.BlockSpec((BN, 1), lambda i: (i, 0)),
        ],
        out_shape=[
            jax.ShapeDtypeStruct((NP, H), jnp.float32),
            jax.ShapeDtypeStruct((NP, 1), jnp.float32),
        ],
    )(gp, h1, dinv, b1r, w2t)


def _k3_body(g2t_ref, z_ref, dinv_ref, b2_ref, out_ref):
    g2t = g2t_ref[...]
    g2 = g2t[:, 0:1] + g2t[:, 1:2]
    out_ref[...] = dinv_ref[...] * (g2 + z_ref[...]) + b2_ref[0]


def _k3(g2t, z, dinv, b2):
    return pl.pallas_call(
        _k3_body,
        grid=(NP // BN,),
        in_specs=[
            pl.BlockSpec((BN, 2), lambda i: (i, 0)),
            pl.BlockSpec((BN, 1), lambda i: (i, 0)),
            pl.BlockSpec((BN, 1), lambda i: (i, 0)),
            pl.BlockSpec(memory_space=pltpu.SMEM),
        ],
        out_specs=pl.BlockSpec((BN, 1), lambda i: (i, 0)),
        out_shape=jax.ShapeDtypeStruct((NP, 1), jnp.float32),
    )(g2t, z, dinv, b2)


@jax.jit
def kernel(x, edge_index, W1, b1, W2, b2):
    src = edge_index[0].astype(jnp.int32)
    dst = edge_index[1].astype(jnp.int32)
    pad = EP - E
    srcp = jnp.concatenate([src, jnp.zeros((pad,), jnp.int32)]).reshape(
        NW, CH, CK
    )
    dstp = jnp.concatenate([dst, jnp.full((pad,), N, jnp.int32)]).reshape(
        NW, CH, CK
    )

    degp = _deg_kernel(dstp)
    xpad = jnp.pad(x, ((0, NP - N), (0, 0)))
    h1, h1p, dinv = _k1(xpad, W1, degp.T)

    gp = _agg_feat_kernel(h1p, srcp, dstp)
    h, z = _k2(gp, h1, dinv, b1.reshape(1, H), W2.reshape(1, H))

    g2p = _agg_scal_kernel(z.reshape(NP), srcp, dstp)
    scores = _k3(g2p.T, z, dinv, b2)

    return h[:N], scores[:N, 0]


# speedup vs baseline: 50.5871x; 1.6023x over previous
import functools
import jax
import jax.numpy as jnp
from jax import lax
from jax.experimental import pallas as pl
from jax.experimental.pallas import tpu as pltpu
from jax.experimental.pallas import tpu_sc as plsc

N = 10000
D = 128
H = 32
E = 320000
NP = 10240
NW = 32
CK = 128
CH = 78
TL = 16
RPT = NP // 16
NBUF = 6

_mesh = plsc.VectorSubcoreMesh(core_axis_name="c", subcore_axis_name="s")
_sc_params = pltpu.CompilerParams(use_tc_tiling_on_sc=False)


def _z16():
    return jnp.zeros((16,), jnp.float32)


def _worker_id():
    return lax.axis_index("s") * 2 + lax.axis_index("c")


@functools.partial(
    pl.kernel,
    out_type=jax.ShapeDtypeStruct((2, NP), jnp.float32),
    mesh=_mesh,
    compiler_params=_sc_params,
    scratch_types=[
        pltpu.VMEM((CH, CK), jnp.int32),
        pltpu.VMEM((TL,), jnp.int32),
        pltpu.VMEM((CK,), jnp.float32),
        pltpu.VMEM((RPT,), jnp.float32),
        pltpu.VMEM_SHARED((NP,), jnp.float32),
        pltpu.SemaphoreType.DMA,
    ],
)
def _deg_kernel(dstm_hbm, dstt_hbm, out_hbm, dst_v, dstt_v, ones_v, zer_v,
                accum, ssem):
    cid = lax.axis_index("c")
    sid = lax.axis_index("s")
    wid = _worker_id()

    def fill_ones(i, _):
        ones_v[pl.ds(i * 16, 16)] = _z16() + 1.0
        return 0

    lax.fori_loop(0, CK // 16, fill_ones, 0)

    def fill_zero(i, _):
        zer_v[pl.ds(i * 16, 16)] = _z16()
        return 0

    lax.fori_loop(0, RPT // 16, fill_zero, 0)

    pltpu.sync_copy(zer_v, accum.at[pl.ds(sid * RPT, RPT)])
    pltpu.sync_copy(dstm_hbm.at[pl.ds(wid * CH, CH)], dst_v)
    pltpu.sync_copy(dstt_hbm.at[wid], dstt_v)
    plsc.subcore_barrier()

    def group(g, _):
        base = g * NBUF
        sd = [
            pltpu.async_copy(
                ones_v, accum.at[dst_v.at[base + b]], ssem, add=True
            )
            for b in range(NBUF)
        ]
        for d in sd:
            d.wait()
        return 0

    lax.fori_loop(0, CH // NBUF, group, 0)
    pltpu.sync_copy(ones_v.at[pl.ds(0, TL)], accum.at[dstt_v], add=True)
    plsc.subcore_barrier()
    pltpu.sync_copy(
        accum.at[pl.ds(sid * RPT, RPT)],
        out_hbm.at[cid, pl.ds(sid * RPT, RPT)],
    )


@functools.partial(
    pl.kernel,
    out_type=jax.ShapeDtypeStruct((2, NP, H), jnp.float32),
    mesh=_mesh,
    compiler_params=_sc_params,
    scratch_types=[
        pltpu.VMEM((CH, CK), jnp.int32),
        pltpu.VMEM((CH, CK), jnp.int32),
        pltpu.VMEM((TL,), jnp.int32),
        pltpu.VMEM((TL,), jnp.int32),
        pltpu.VMEM((NBUF, CK, H), jnp.float32),
        pltpu.VMEM((TL, H), jnp.float32),
        pltpu.VMEM_SHARED((NP, H), jnp.float32),
        pltpu.SemaphoreType.DMA,
        pltpu.SemaphoreType.DMA,
    ],
)
def _agg_feat_kernel(tbl_hbm, srcm_hbm, dstm_hbm, srct_hbm, dstt_hbm, out_hbm,
                     src_v, dst_v, srct_v, dstt_v, rows_v, rowt_v,
                     accum, gsem, ssem):
    cid = lax.axis_index("c")
    sid = lax.axis_index("s")
    wid = _worker_id()

    def fill_zero(i, _):
        rows_v[0, i, pl.ds(0, 16)] = _z16()
        rows_v[0, i, pl.ds(16, 16)] = _z16()
        return 0

    lax.fori_loop(0, CK, fill_zero, 0)

    def zcopy(i, _):
        pltpu.sync_copy(
            rows_v.at[0], accum.at[pl.ds(sid * RPT + i * CK, CK)]
        )
        return 0

    lax.fori_loop(0, RPT // CK, zcopy, 0)
    pltpu.sync_copy(srcm_hbm.at[pl.ds(wid * CH, CH)], src_v)
    pltpu.sync_copy(dstm_hbm.at[pl.ds(wid * CH, CH)], dst_v)
    pltpu.sync_copy(srct_hbm.at[wid], srct_v)
    pltpu.sync_copy(dstt_hbm.at[wid], dstt_v)
    plsc.subcore_barrier()

    def group(g, _):
        base = g * NBUF
        gd = [
            pltpu.async_copy(
                tbl_hbm.at[src_v.at[base + b]], rows_v.at[b], gsem
            )
            for b in range(NBUF)
        ]
        for d in gd:
            d.wait()
        sd = [
            pltpu.async_copy(
                rows_v.at[b], accum.at[dst_v.at[base + b]], ssem, add=True
            )
            for b in range(NBUF)
        ]
        for d in sd:
            d.wait()
        return 0

    lax.fori_loop(0, CH // NBUF, group, 0)
    pltpu.async_copy(tbl_hbm.at[srct_v], rowt_v, gsem).wait()
    pltpu.sync_copy(rowt_v, accum.at[dstt_v], add=True)
    plsc.subcore_barrier()
    pltpu.sync_copy(
        accum.at[pl.ds(sid * RPT, RPT)],
        out_hbm.at[cid, pl.ds(sid * RPT, RPT)],
    )


@functools.partial(
    pl.kernel,
    out_type=jax.ShapeDtypeStruct((2, NP), jnp.float32),
    mesh=_mesh,
    compiler_params=_sc_params,
    scratch_types=[
        pltpu.VMEM((CH, CK), jnp.int32),
        pltpu.VMEM((CH, CK), jnp.int32),
        pltpu.VMEM((TL,), jnp.int32),
        pltpu.VMEM((TL,), jnp.int32),
        pltpu.VMEM((NBUF, CK), jnp.float32),
        pltpu.VMEM((TL,), jnp.float32),
        pltpu.VMEM((RPT,), jnp.float32),
        pltpu.VMEM_SHARED((NP,), jnp.float32),
        pltpu.SemaphoreType.DMA,
        pltpu.SemaphoreType.DMA,
    ],
)
def _agg_scal_kernel(tbl_hbm, srcm_hbm, dstm_hbm, srct_hbm, dstt_hbm, out_hbm,
                     src_v, dst_v, srct_v, dstt_v, rows_v, rowt_v, zer_v,
                     accum, gsem, ssem):
    cid = lax.axis_index("c")
    sid = lax.axis_index("s")
    wid = _worker_id()

    def fill_zero(i, _):
        zer_v[pl.ds(i * 16, 16)] = _z16()
        return 0

    lax.fori_loop(0, RPT // 16, fill_zero, 0)

    pltpu.sync_copy(zer_v, accum.at[pl.ds(sid * RPT, RPT)])
    pltpu.sync_copy(srcm_hbm.at[pl.ds(wid * CH, CH)], src_v)
    pltpu.sync_copy(dstm_hbm.at[pl.ds(wid * CH, CH)], dst_v)
    pltpu.sync_copy(srct_hbm.at[wid], srct_v)
    pltpu.sync_copy(dstt_hbm.at[wid], dstt_v)
    plsc.subcore_barrier()

    def group(g, _):
        base = g * NBUF
        gd = [
            pltpu.async_copy(
                tbl_hbm.at[src_v.at[base + b]], rows_v.at[b], gsem
            )
            for b in range(NBUF)
        ]
        for d in gd:
            d.wait()
        sd = [
            pltpu.async_copy(
                rows_v.at[b], accum.at[dst_v.at[base + b]], ssem, add=True
            )
            for b in range(NBUF)
        ]
        for d in sd:
            d.wait()
        return 0

    lax.fori_loop(0, CH // NBUF, group, 0)
    pltpu.async_copy(tbl_hbm.at[srct_v], rowt_v, gsem).wait()
    pltpu.sync_copy(rowt_v, accum.at[dstt_v], add=True)
    plsc.subcore_barrier()
    pltpu.sync_copy(
        accum.at[pl.ds(sid * RPT, RPT)],
        out_hbm.at[cid, pl.ds(sid * RPT, RPT)],
    )


BN = 1000


def _k1_body(x_ref, w1_ref, degt_ref, h1_ref, h1p_ref, dinv_ref):
    h1 = jnp.dot(x_ref[...], w1_ref[...], preferred_element_type=jnp.float32)
    degt = degt_ref[...]
    deg = degt[:, 0:1] + degt[:, 1:2] + 1.0
    dinv = lax.rsqrt(deg)
    h1_ref[...] = h1
    h1p_ref[...] = h1 * dinv
    dinv_ref[...] = dinv


def _k1(x, W1, degt):
    return pl.pallas_call(
        _k1_body,
        grid=(N // BN,),
        in_specs=[
            pl.BlockSpec((BN, D), lambda i: (i, 0)),
            pl.BlockSpec((D, H), lambda i: (0, 0)),
            pl.BlockSpec((BN, 2), lambda i: (i, 0)),
        ],
        out_specs=[
            pl.BlockSpec((BN, H), lambda i: (i, 0)),
            pl.BlockSpec((BN, H), lambda i: (i, 0)),
            pl.BlockSpec((BN, 1), lambda i: (i, 0)),
        ],
        out_shape=[
            jax.ShapeDtypeStruct((N, H), jnp.float32),
            jax.ShapeDtypeStruct((N, H), jnp.float32),
            jax.ShapeDtypeStruct((N, 1), jnp.float32),
        ],
    )(x, W1, degt)


def _k2_body(gp_ref, h1_ref, dinv_ref, b1_ref, w2t_ref, h_ref, z_ref):
    g = gp_ref[0] + gp_ref[1]
    dinv = dinv_ref[...]
    acc = dinv * (g + dinv * h1_ref[...]) + b1_ref[...]
    h = jnp.maximum(acc, 0.0)
    h_ref[...] = h
    h2 = jnp.sum(h * w2t_ref[...], axis=1, keepdims=True)
    z_ref[...] = dinv * h2


def _k2(gp, h1, dinv, b1r, w2t):
    return pl.pallas_call(
        _k2_body,
        grid=(N // BN,),
        in_specs=[
            pl.BlockSpec((2, BN, H), lambda i: (0, i, 0)),
            pl.BlockSpec((BN, H), lambda i: (i, 0)),
            pl.BlockSpec((BN, 1), lambda i: (i, 0)),
            pl.BlockSpec((1, H), lambda i: (0, 0)),
            pl.BlockSpec((1, H), lambda i: (0, 0)),
        ],
        out_specs=[
            pl.BlockSpec((BN, H), lambda i: (i, 0)),
            pl.BlockSpec((BN, 1), lambda i: (i, 0)),
        ],
        out_shape=[
            jax.ShapeDtypeStruct((N, H), jnp.float32),
            jax.ShapeDtypeStruct((N, 1), jnp.float32),
        ],
    )(gp, h1, dinv, b1r, w2t)


def _k3_body(g2t_ref, z_ref, dinv_ref, b2_ref, out_ref):
    g2t = g2t_ref[...]
    g2 = g2t[:, 0:1] + g2t[:, 1:2]
    out_ref[...] = dinv_ref[...] * (g2 + z_ref[...]) + b2_ref[0]


def _k3(g2t, z, dinv, b2):
    return pl.pallas_call(
        _k3_body,
        grid=(N // BN,),
        in_specs=[
            pl.BlockSpec((BN, 2), lambda i: (i, 0)),
            pl.BlockSpec((BN, 1), lambda i: (i, 0)),
            pl.BlockSpec((BN, 1), lambda i: (i, 0)),
            pl.BlockSpec(memory_space=pltpu.SMEM),
        ],
        out_specs=pl.BlockSpec((BN, 1), lambda i: (i, 0)),
        out_shape=jax.ShapeDtypeStruct((N, 1), jnp.float32),
    )(g2t, z, dinv, b2)


EM = NW * CH * CK


@jax.jit
def kernel(x, edge_index, W1, b1, W2, b2):
    ei = edge_index.astype(jnp.int32)
    srcm = ei[0, :EM].reshape(NW * CH, CK)
    dstm = ei[1, :EM].reshape(NW * CH, CK)
    srct = ei[0, EM:].reshape(NW, TL)
    dstt = ei[1, EM:].reshape(NW, TL)

    degp = _deg_kernel(dstm, dstt)
    h1, h1p, dinv = _k1(x, W1, degp[:, :N].T)

    gp = _agg_feat_kernel(h1p, srcm, dstm, srct, dstt)
    h, z = _k2(gp[:, :N], h1, dinv, b1.reshape(1, H), W2.reshape(1, H))

    g2p = _agg_scal_kernel(z.reshape(N), srcm, dstm, srct, dstt)
    scores = _k3(g2p[:, :N].T, z, dinv, b2)

    return h, scores[:, 0]


# speedup vs baseline: 58.7655x; 1.1617x over previous
import functools
import jax
import jax.numpy as jnp
from jax import lax
from jax.experimental import pallas as pl
from jax.experimental.pallas import tpu as pltpu
from jax.experimental.pallas import tpu_sc as plsc

N = 10000
D = 128
H = 32
E = 320000
NP = 10240
NW = 32
CK = 128
CH = 78
TL = 16
RPT = NP // 16
NBUF = 6

_mesh = plsc.VectorSubcoreMesh(core_axis_name="c", subcore_axis_name="s")
_sc_params = pltpu.CompilerParams(use_tc_tiling_on_sc=False)
_sc_params_nl = pltpu.CompilerParams(
    use_tc_tiling_on_sc=False, needs_layout_passes=False
)


def _z16():
    return jnp.zeros((16,), jnp.float32)


def _worker_id():
    return lax.axis_index("s") * 2 + lax.axis_index("c")


@functools.partial(
    pl.kernel,
    out_type=jax.ShapeDtypeStruct((2, NP), jnp.float32),
    mesh=_mesh,
    compiler_params=_sc_params,
    scratch_types=[
        pltpu.VMEM((CH, CK), jnp.int32),
        pltpu.VMEM((TL,), jnp.int32),
        pltpu.VMEM((CK,), jnp.float32),
        pltpu.VMEM((RPT,), jnp.float32),
        pltpu.VMEM_SHARED((NP,), jnp.float32),
        pltpu.SemaphoreType.DMA,
    ],
)
def _deg_kernel(dstm_hbm, dstt_hbm, out_hbm, dst_v, dstt_v, ones_v, zer_v,
                accum, ssem):
    cid = lax.axis_index("c")
    sid = lax.axis_index("s")
    wid = _worker_id()

    def fill_ones(i, _):
        ones_v[pl.ds(i * 16, 16)] = _z16() + 1.0
        return 0

    lax.fori_loop(0, CK // 16, fill_ones, 0)

    def fill_zero(i, _):
        zer_v[pl.ds(i * 16, 16)] = _z16()
        return 0

    lax.fori_loop(0, RPT // 16, fill_zero, 0)

    pltpu.sync_copy(zer_v, accum.at[pl.ds(sid * RPT, RPT)])
    pltpu.sync_copy(dstm_hbm.at[pl.ds(wid * CH, CH)], dst_v)
    pltpu.sync_copy(dstt_hbm.at[wid], dstt_v)
    plsc.subcore_barrier()

    def group(g, _):
        base = g * NBUF
        sd = [
            pltpu.async_copy(
                ones_v, accum.at[dst_v.at[base + b]], ssem, add=True
            )
            for b in range(NBUF)
        ]
        for d in sd:
            d.wait()
        return 0

    lax.fori_loop(0, CH // NBUF, group, 0)
    pltpu.sync_copy(ones_v.at[pl.ds(0, TL)], accum.at[dstt_v], add=True)
    plsc.subcore_barrier()
    pltpu.sync_copy(
        accum.at[pl.ds(sid * RPT, RPT)],
        out_hbm.at[cid, pl.ds(sid * RPT, RPT)],
    )


@functools.partial(
    pl.kernel,
    out_type=jax.ShapeDtypeStruct((2, NP, H), jnp.float32),
    mesh=_mesh,
    compiler_params=_sc_params,
    scratch_types=[
        pltpu.VMEM((CH, CK), jnp.int32),
        pltpu.VMEM((CH, CK), jnp.int32),
        pltpu.VMEM((TL,), jnp.int32),
        pltpu.VMEM((TL,), jnp.int32),
        pltpu.VMEM((NBUF, CK, H), jnp.float32),
        pltpu.VMEM((TL, H), jnp.float32),
        pltpu.VMEM_SHARED((NP, H), jnp.float32),
        pltpu.SemaphoreType.DMA,
        pltpu.SemaphoreType.DMA,
    ],
)
def _agg_feat_kernel(tbl_hbm, srcm_hbm, dstm_hbm, srct_hbm, dstt_hbm, out_hbm,
                     src_v, dst_v, srct_v, dstt_v, rows_v, rowt_v,
                     accum, gsem, ssem):
    cid = lax.axis_index("c")
    sid = lax.axis_index("s")
    wid = _worker_id()

    def fill_zero(i, _):
        rows_v[0, i, pl.ds(0, 16)] = _z16()
        rows_v[0, i, pl.ds(16, 16)] = _z16()
        return 0

    lax.fori_loop(0, CK, fill_zero, 0)

    def zcopy(i, _):
        pltpu.sync_copy(
            rows_v.at[0], accum.at[pl.ds(sid * RPT + i * CK, CK)]
        )
        return 0

    lax.fori_loop(0, RPT // CK, zcopy, 0)
    pltpu.sync_copy(srcm_hbm.at[pl.ds(wid * CH, CH)], src_v)
    pltpu.sync_copy(dstm_hbm.at[pl.ds(wid * CH, CH)], dst_v)
    pltpu.sync_copy(srct_hbm.at[wid], srct_v)
    pltpu.sync_copy(dstt_hbm.at[wid], dstt_v)
    plsc.subcore_barrier()

    def group(g, _):
        base = g * NBUF
        gd = [
            pltpu.async_copy(
                tbl_hbm.at[src_v.at[base + b]], rows_v.at[b], gsem
            )
            for b in range(NBUF)
        ]
        for d in gd:
            d.wait()
        sd = [
            pltpu.async_copy(
                rows_v.at[b], accum.at[dst_v.at[base + b]], ssem, add=True
            )
            for b in range(NBUF)
        ]
        for d in sd:
            d.wait()
        return 0

    lax.fori_loop(0, CH // NBUF, group, 0)
    pltpu.async_copy(tbl_hbm.at[srct_v], rowt_v, gsem).wait()
    pltpu.sync_copy(rowt_v, accum.at[dstt_v], add=True)
    plsc.subcore_barrier()
    pltpu.sync_copy(
        accum.at[pl.ds(sid * RPT, RPT)],
        out_hbm.at[cid, pl.ds(sid * RPT, RPT)],
    )


@functools.partial(
    pl.kernel,
    out_type=jax.ShapeDtypeStruct((2, NP), jnp.float32),
    mesh=_mesh,
    compiler_params=_sc_params_nl,
    scratch_types=[
        pltpu.VMEM((CH, CK), jnp.int32),
        pltpu.VMEM((CH, CK), jnp.int32),
        pltpu.VMEM((TL,), jnp.int32),
        pltpu.VMEM((TL,), jnp.int32),
        pltpu.VMEM((N,), jnp.float32),
        pltpu.VMEM((NP,), jnp.float32),
        pltpu.VMEM((16, RPT), jnp.float32),
        pltpu.VMEM_SHARED((16, NP), jnp.float32),
    ],
)
def _agg_scal_kernel(tbl_hbm, srcm_hbm, dstm_hbm, srct_hbm, dstt_hbm, out_hbm,
                     src_v, dst_v, srct_v, dstt_v, z_v, acc_v, red_v, stage):
    cid = lax.axis_index("c")
    sid = lax.axis_index("s")
    wid = _worker_id()

    pltpu.sync_copy(tbl_hbm, z_v)
    pltpu.sync_copy(srcm_hbm.at[pl.ds(wid * CH, CH)], src_v)
    pltpu.sync_copy(dstm_hbm.at[pl.ds(wid * CH, CH)], dst_v)
    pltpu.sync_copy(srct_hbm.at[wid], srct_v)
    pltpu.sync_copy(dstt_hbm.at[wid], dstt_v)

    def zero(i, _):
        acc_v[pl.ds(i * 16, 16)] = _z16()
        return 0

    lax.fori_loop(0, NP // 16, zero, 0)

    def chunk(i, _):
        def sub(j, _):
            s_idx = src_v[i, pl.ds(j * 16, 16)]
            d_idx = dst_v[i, pl.ds(j * 16, 16)]
            vals = plsc.load_gather(z_v, [s_idx])
            plsc.addupdate_scatter(acc_v, [d_idx], vals)
            return 0

        lax.fori_loop(0, CK // 16, sub, 0)
        return 0

    lax.fori_loop(0, CH, chunk, 0)
    vals = plsc.load_gather(z_v, [srct_v[...]])
    plsc.addupdate_scatter(acc_v, [dstt_v[...]], vals)

    pltpu.sync_copy(acc_v, stage.at[sid])
    plsc.subcore_barrier()
    for t in range(16):
        pltpu.sync_copy(stage.at[t, pl.ds(sid * RPT, RPT)], red_v.at[t])

    def redsum(r, _):
        s = red_v[0, pl.ds(r * 16, 16)]
        for t in range(1, 16):
            s = s + red_v[t, pl.ds(r * 16, 16)]
        acc_v[pl.ds(r * 16, 16)] = s
        return 0

    lax.fori_loop(0, RPT // 16, redsum, 0)
    pltpu.sync_copy(
        acc_v.at[pl.ds(0, RPT)],
        out_hbm.at[cid, pl.ds(sid * RPT, RPT)],
    )


BN = 1000


def _k1_body(x_ref, w1_ref, degt_ref, h1_ref, h1p_ref, dinv_ref):
    h1 = jnp.dot(x_ref[...], w1_ref[...], preferred_element_type=jnp.float32)
    degt = degt_ref[...]
    deg = degt[:, 0:1] + degt[:, 1:2] + 1.0
    dinv = lax.rsqrt(deg)
    h1_ref[...] = h1
    h1p_ref[...] = h1 * dinv
    dinv_ref[...] = dinv


def _k1(x, W1, degt):
    return pl.pallas_call(
        _k1_body,
        grid=(N // BN,),
        in_specs=[
            pl.BlockSpec((BN, D), lambda i: (i, 0)),
            pl.BlockSpec((D, H), lambda i: (0, 0)),
            pl.BlockSpec((BN, 2), lambda i: (i, 0)),
        ],
        out_specs=[
            pl.BlockSpec((BN, H), lambda i: (i, 0)),
            pl.BlockSpec((BN, H), lambda i: (i, 0)),
            pl.BlockSpec((BN, 1), lambda i: (i, 0)),
        ],
        out_shape=[
            jax.ShapeDtypeStruct((N, H), jnp.float32),
            jax.ShapeDtypeStruct((N, H), jnp.float32),
            jax.ShapeDtypeStruct((N, 1), jnp.float32),
        ],
    )(x, W1, degt)


def _k2_body(gp_ref, h1_ref, dinv_ref, b1_ref, w2t_ref, h_ref, z_ref):
    g = gp_ref[0] + gp_ref[1]
    dinv = dinv_ref[...]
    acc = dinv * (g + dinv * h1_ref[...]) + b1_ref[...]
    h = jnp.maximum(acc, 0.0)
    h_ref[...] = h
    h2 = jnp.sum(h * w2t_ref[...], axis=1, keepdims=True)
    z_ref[...] = dinv * h2


def _k2(gp, h1, dinv, b1r, w2t):
    return pl.pallas_call(
        _k2_body,
        grid=(N // BN,),
        in_specs=[
            pl.BlockSpec((2, BN, H), lambda i: (0, i, 0)),
            pl.BlockSpec((BN, H), lambda i: (i, 0)),
            pl.BlockSpec((BN, 1), lambda i: (i, 0)),
            pl.BlockSpec((1, H), lambda i: (0, 0)),
            pl.BlockSpec((1, H), lambda i: (0, 0)),
        ],
        out_specs=[
            pl.BlockSpec((BN, H), lambda i: (i, 0)),
            pl.BlockSpec((BN, 1), lambda i: (i, 0)),
        ],
        out_shape=[
            jax.ShapeDtypeStruct((N, H), jnp.float32),
            jax.ShapeDtypeStruct((N, 1), jnp.float32),
        ],
    )(gp, h1, dinv, b1r, w2t)


def _k3_body(g2t_ref, z_ref, dinv_ref, b2_ref, out_ref):
    g2t = g2t_ref[...]
    g2 = g2t[:, 0:1] + g2t[:, 1:2]
    out_ref[...] = dinv_ref[...] * (g2 + z_ref[...]) + b2_ref[0]


def _k3(g2t, z, dinv, b2):
    return pl.pallas_call(
        _k3_body,
        grid=(N // BN,),
        in_specs=[
            pl.BlockSpec((BN, 2), lambda i: (i, 0)),
            pl.BlockSpec((BN, 1), lambda i: (i, 0)),
            pl.BlockSpec((BN, 1), lambda i: (i, 0)),
            pl.BlockSpec(memory_space=pltpu.SMEM),
        ],
        out_specs=pl.BlockSpec((BN, 1), lambda i: (i, 0)),
        out_shape=jax.ShapeDtypeStruct((N, 1), jnp.float32),
    )(g2t, z, dinv, b2)


EM = NW * CH * CK


@jax.jit
def kernel(x, edge_index, W1, b1, W2, b2):
    ei = edge_index.astype(jnp.int32)
    srcm = ei[0, :EM].reshape(NW * CH, CK)
    dstm = ei[1, :EM].reshape(NW * CH, CK)
    srct = ei[0, EM:].reshape(NW, TL)
    dstt = ei[1, EM:].reshape(NW, TL)

    degp = _deg_kernel(dstm, dstt)
    h1, h1p, dinv = _k1(x, W1, degp[:, :N].T)

    gp = _agg_feat_kernel(h1p, srcm, dstm, srct, dstt)
    h, z = _k2(gp[:, :N], h1, dinv, b1.reshape(1, H), W2.reshape(1, H))

    g2p = _agg_scal_kernel(z.reshape(N), srcm, dstm, srct, dstt)
    scores = _k3(g2p[:, :N].T, z, dinv, b2)

    return h, scores[:, 0]


# speedup vs baseline: 73.7376x; 1.2548x over previous
import functools
import jax
import jax.numpy as jnp
from jax import lax
from jax.experimental import pallas as pl
from jax.experimental.pallas import tpu as pltpu
from jax.experimental.pallas import tpu_sc as plsc

N = 10000
D = 128
H = 32
E = 320000
NP = 10240
NW = 32
CK = 128
EC = E // CK
CH = 78
XW = EC - NW * CH
RPT = NP // 16
NBUF = 6

_mesh = plsc.VectorSubcoreMesh(core_axis_name="c", subcore_axis_name="s")
_sc_params = pltpu.CompilerParams(use_tc_tiling_on_sc=False)
_sc_params_nl = pltpu.CompilerParams(
    use_tc_tiling_on_sc=False, needs_layout_passes=False
)


def _z16():
    return jnp.zeros((16,), jnp.float32)


def _worker_id():
    return lax.axis_index("s") * 2 + lax.axis_index("c")


def _chunk_base(wid):
    return jnp.where(wid < XW, (CH + 1) * wid, XW + CH * wid)


@functools.partial(
    pl.kernel,
    out_type=jax.ShapeDtypeStruct((2, NP), jnp.float32),
    mesh=_mesh,
    compiler_params=_sc_params,
    scratch_types=[
        pltpu.VMEM((CH + 1, CK), jnp.int32),
        pltpu.VMEM((CK,), jnp.float32),
        pltpu.VMEM((RPT,), jnp.float32),
        pltpu.VMEM_SHARED((NP,), jnp.float32),
        pltpu.SemaphoreType.DMA,
    ],
)
def _deg_kernel(ei_hbm, out_hbm, dst_v, ones_v, zer_v, accum, ssem):
    cid = lax.axis_index("c")
    sid = lax.axis_index("s")
    wid = _worker_id()
    base = _chunk_base(wid)

    def fill_ones(i, _):
        ones_v[pl.ds(i * 16, 16)] = _z16() + 1.0
        return 0

    lax.fori_loop(0, CK // 16, fill_ones, 0)

    def fill_zero(i, _):
        zer_v[pl.ds(i * 16, 16)] = _z16()
        return 0

    lax.fori_loop(0, RPT // 16, fill_zero, 0)

    pltpu.sync_copy(zer_v, accum.at[pl.ds(sid * RPT, RPT)])
    pltpu.sync_copy(ei_hbm.at[1, pl.ds(base, CH)], dst_v.at[pl.ds(0, CH)])

    @pl.when(wid < XW)
    def _():
        pltpu.sync_copy(ei_hbm.at[1, base + CH], dst_v.at[CH])

    plsc.subcore_barrier()

    def group(g, _):
        gb = g * NBUF
        sd = [
            pltpu.async_copy(
                ones_v, accum.at[dst_v.at[gb + b]], ssem, add=True
            )
            for b in range(NBUF)
        ]
        for d in sd:
            d.wait()
        return 0

    lax.fori_loop(0, CH // NBUF, group, 0)

    @pl.when(wid < XW)
    def _():
        pltpu.sync_copy(ones_v, accum.at[dst_v.at[CH]], add=True)

    plsc.subcore_barrier()
    pltpu.sync_copy(
        accum.at[pl.ds(sid * RPT, RPT)],
        out_hbm.at[cid, pl.ds(sid * RPT, RPT)],
    )


@functools.partial(
    pl.kernel,
    out_type=jax.ShapeDtypeStruct((2, NP, H), jnp.float32),
    mesh=_mesh,
    compiler_params=_sc_params,
    scratch_types=[
        pltpu.VMEM((CH + 1, CK), jnp.int32),
        pltpu.VMEM((CH + 1, CK), jnp.int32),
        pltpu.VMEM((NBUF, CK, H), jnp.float32),
        pltpu.VMEM_SHARED((NP, H), jnp.float32),
        pltpu.SemaphoreType.DMA,
        pltpu.SemaphoreType.DMA,
    ],
)
def _agg_feat_kernel(tbl_hbm, ei_hbm, out_hbm, src_v, dst_v, rows_v,
                     accum, gsem, ssem):
    cid = lax.axis_index("c")
    sid = lax.axis_index("s")
    wid = _worker_id()
    base = _chunk_base(wid)

    def fill_zero(i, _):
        rows_v[0, i, pl.ds(0, 16)] = _z16()
        rows_v[0, i, pl.ds(16, 16)] = _z16()
        return 0

    lax.fori_loop(0, CK, fill_zero, 0)

    def zcopy(i, _):
        pltpu.sync_copy(
            rows_v.at[0], accum.at[pl.ds(sid * RPT + i * CK, CK)]
        )
        return 0

    lax.fori_loop(0, RPT // CK, zcopy, 0)
    pltpu.sync_copy(ei_hbm.at[0, pl.ds(base, CH)], src_v.at[pl.ds(0, CH)])
    pltpu.sync_copy(ei_hbm.at[1, pl.ds(base, CH)], dst_v.at[pl.ds(0, CH)])

    @pl.when(wid < XW)
    def _():
        pltpu.sync_copy(ei_hbm.at[0, base + CH], src_v.at[CH])
        pltpu.sync_copy(ei_hbm.at[1, base + CH], dst_v.at[CH])

    plsc.subcore_barrier()

    def group(g, _):
        gb = g * NBUF
        gd = [
            pltpu.async_copy(
                tbl_hbm.at[src_v.at[gb + b]], rows_v.at[b], gsem
            )
            for b in range(NBUF)
        ]
        for d in gd:
            d.wait()
        sd = [
            pltpu.async_copy(
                rows_v.at[b], accum.at[dst_v.at[gb + b]], ssem, add=True
            )
            for b in range(NBUF)
        ]
        for d in sd:
            d.wait()
        return 0

    lax.fori_loop(0, CH // NBUF, group, 0)

    @pl.when(wid < XW)
    def _():
        pltpu.async_copy(tbl_hbm.at[src_v.at[CH]], rows_v.at[0], gsem).wait()
        pltpu.sync_copy(rows_v.at[0], accum.at[dst_v.at[CH]], add=True)

    plsc.subcore_barrier()
    pltpu.sync_copy(
        accum.at[pl.ds(sid * RPT, RPT)],
        out_hbm.at[cid, pl.ds(sid * RPT, RPT)],
    )


@functools.partial(
    pl.kernel,
    out_type=jax.ShapeDtypeStruct((2, NP), jnp.float32),
    mesh=_mesh,
    compiler_params=_sc_params_nl,
    scratch_types=[
        pltpu.VMEM((CH + 1, CK), jnp.int32),
        pltpu.VMEM((CH + 1, CK), jnp.int32),
        pltpu.VMEM((NP,), jnp.float32),
        pltpu.VMEM((NP,), jnp.float32),
        pltpu.VMEM((16, RPT), jnp.float32),
        pltpu.VMEM_SHARED((16, NP), jnp.float32),
    ],
)
def _agg_scal_kernel(tbl_hbm, ei_hbm, out_hbm, src_v, dst_v, z_v, acc_v,
                     red_v, stage):
    cid = lax.axis_index("c")
    sid = lax.axis_index("s")
    wid = _worker_id()
    base = _chunk_base(wid)

    pltpu.sync_copy(tbl_hbm, z_v)
    pltpu.sync_copy(ei_hbm.at[0, pl.ds(base, CH)], src_v.at[pl.ds(0, CH)])
    pltpu.sync_copy(ei_hbm.at[1, pl.ds(base, CH)], dst_v.at[pl.ds(0, CH)])

    @pl.when(wid < XW)
    def _():
        pltpu.sync_copy(ei_hbm.at[0, base + CH], src_v.at[CH])
        pltpu.sync_copy(ei_hbm.at[1, base + CH], dst_v.at[CH])

    def zero(i, _):
        acc_v[pl.ds(i * 16, 16)] = _z16()
        return 0

    lax.fori_loop(0, NP // 16, zero, 0)

    def chunk(i, _):
        def sub(j, _):
            s_idx = src_v[i, pl.ds(j * 16, 16)]
            d_idx = dst_v[i, pl.ds(j * 16, 16)]
            vals = plsc.load_gather(z_v, [s_idx])
            plsc.addupdate_scatter(acc_v, [d_idx], vals)
            return 0

        lax.fori_loop(0, CK // 16, sub, 0)
        return 0

    lax.fori_loop(0, CH, chunk, 0)

    @pl.when(wid < XW)
    def _():
        def sub(j, _):
            s_idx = src_v[CH, pl.ds(j * 16, 16)]
            d_idx = dst_v[CH, pl.ds(j * 16, 16)]
            vals = plsc.load_gather(z_v, [s_idx])
            plsc.addupdate_scatter(acc_v, [d_idx], vals)
            return 0

        lax.fori_loop(0, CK // 16, sub, 0)

    pltpu.sync_copy(acc_v, stage.at[sid])
    plsc.subcore_barrier()
    for t in range(16):
        pltpu.sync_copy(stage.at[t, pl.ds(sid * RPT, RPT)], red_v.at[t])

    def redsum(r, _):
        s = red_v[0, pl.ds(r * 16, 16)]
        for t in range(1, 16):
            s = s + red_v[t, pl.ds(r * 16, 16)]
        acc_v[pl.ds(r * 16, 16)] = s
        return 0

    lax.fori_loop(0, RPT // 16, redsum, 0)
    pltpu.sync_copy(
        acc_v.at[pl.ds(0, RPT)],
        out_hbm.at[cid, pl.ds(sid * RPT, RPT)],
    )


BN = 1024


def _k1_body(x_ref, w1t_ref, degp_ref, h1_ref, h1p_ref, dinv_ref):
    h1 = lax.dot_general(
        w1t_ref[...], x_ref[...], (((1,), (1,)), ((), ())),
        preferred_element_type=jnp.float32,
    )
    degp = degp_ref[...]
    deg = degp[0:1, :] + degp[1:2, :] + 1.0
    dinv = lax.rsqrt(deg)
    h1_ref[...] = h1
    h1p_ref[...] = h1 * dinv
    dinv_ref[...] = dinv


def _k1(xp, w1t, degp):
    return pl.pallas_call(
        _k1_body,
        grid=(NP // BN,),
        in_specs=[
            pl.BlockSpec((BN, D), lambda i: (i, 0)),
            pl.BlockSpec((H, D), lambda i: (0, 0)),
            pl.BlockSpec((2, BN), lambda i: (0, i)),
        ],
        out_specs=[
            pl.BlockSpec((H, BN), lambda i: (0, i)),
            pl.BlockSpec((H, BN), lambda i: (0, i)),
            pl.BlockSpec((1, BN), lambda i: (0, i)),
        ],
        out_shape=[
            jax.ShapeDtypeStruct((H, NP), jnp.float32),
            jax.ShapeDtypeStruct((H, NP), jnp.float32),
            jax.ShapeDtypeStruct((1, NP), jnp.float32),
        ],
    )(xp, w1t, degp)


def _k2_body(gt_ref, h1_ref, dinv_ref, b1c_ref, w2r_ref, h_ref, z_ref):
    g = gt_ref[0] + gt_ref[1]
    dinv = dinv_ref[...]
    acc = dinv * (g + dinv * h1_ref[...]) + b1c_ref[...]
    h = jnp.maximum(acc, 0.0)
    h_ref[...] = h
    h2 = lax.dot_general(
        w2r_ref[...], h, (((1,), (0,)), ((), ())),
        preferred_element_type=jnp.float32,
    )
    z_ref[...] = dinv * h2


def _k2(gt, h1, dinv, b1c, w2r):
    return pl.pallas_call(
        _k2_body,
        grid=(NP // BN,),
        in_specs=[
            pl.BlockSpec((2, H, BN), lambda i: (0, 0, i)),
            pl.BlockSpec((H, BN), lambda i: (0, i)),
            pl.BlockSpec((1, BN), lambda i: (0, i)),
            pl.BlockSpec((H, 1), lambda i: (0, 0)),
            pl.BlockSpec((1, H), lambda i: (0, 0)),
        ],
        out_specs=[
            pl.BlockSpec((H, BN), lambda i: (0, i)),
            pl.BlockSpec((1, BN), lambda i: (0, i)),
        ],
        out_shape=[
            jax.ShapeDtypeStruct((H, NP), jnp.float32),
            jax.ShapeDtypeStruct((1, NP), jnp.float32),
        ],
    )(gt, h1, dinv, b1c, w2r)


def _k3_body(g2p_ref, z_ref, dinv_ref, b2_ref, out_ref):
    g2p = g2p_ref[...]
    g2 = g2p[0:1, :] + g2p[1:2, :]
    out_ref[...] = dinv_ref[...] * (g2 + z_ref[...]) + b2_ref[0]


def _k3(g2p, z, dinv, b2):
    return pl.pallas_call(
        _k3_body,
        in_specs=[
            pl.BlockSpec((2, NP), lambda: (0, 0)),
            pl.BlockSpec((1, NP), lambda: (0, 0)),
            pl.BlockSpec((1, NP), lambda: (0, 0)),
            pl.BlockSpec(memory_space=pltpu.SMEM),
        ],
        out_specs=pl.BlockSpec((1, NP), lambda: (0, 0)),
        out_shape=jax.ShapeDtypeStruct((1, NP), jnp.float32),
    )(g2p, z, dinv, b2)


@jax.jit
def kernel(x, edge_index, W1, b1, W2, b2):
    eir = edge_index.astype(jnp.int32).reshape(2, EC, CK)

    degp = _deg_kernel(eir)
    xp = jnp.pad(x, ((0, NP - N), (0, 0)))
    h1T, h1pT, dinv = _k1(xp, W1.T, degp)

    gp = _agg_feat_kernel(h1pT.T, eir)
    gt = jnp.transpose(gp, (0, 2, 1))
    hT, z = _k2(gt, h1T, dinv, b1.reshape(H, 1), W2.reshape(1, H))

    g2p = _agg_scal_kernel(z.reshape(NP), eir)
    scores = _k3(g2p, z, dinv, b2)

    return hT.T[:N], scores[0, :N]


# speedup vs baseline: 76.5848x; 1.0386x over previous
import functools
import jax
import jax.numpy as jnp
from jax import lax
from jax.experimental import pallas as pl
from jax.experimental.pallas import tpu as pltpu
from jax.experimental.pallas import tpu_sc as plsc

N = 10000
D = 128
H = 32
E = 320000
NP = 10240
NW = 32
CK = 128
EC = E // CK
CH = 78
XW = EC - NW * CH
RPT = NP // 16
NBUF = 6

_mesh = plsc.VectorSubcoreMesh(core_axis_name="c", subcore_axis_name="s")
_sc_params = pltpu.CompilerParams(use_tc_tiling_on_sc=False)
_sc_params_nl = pltpu.CompilerParams(
    use_tc_tiling_on_sc=False, needs_layout_passes=False
)


def _z16():
    return jnp.zeros((16,), jnp.float32)


def _worker_id():
    return lax.axis_index("s") * 2 + lax.axis_index("c")


def _chunk_base(wid):
    return jnp.where(wid < XW, (CH + 1) * wid, XW + CH * wid)


@functools.partial(
    pl.kernel,
    out_type=jax.ShapeDtypeStruct((2, NP), jnp.float32),
    mesh=_mesh,
    compiler_params=_sc_params,
    scratch_types=[
        pltpu.VMEM((CH + 1, CK), jnp.int32),
        pltpu.VMEM((CK,), jnp.float32),
        pltpu.VMEM((RPT,), jnp.float32),
        pltpu.VMEM_SHARED((NP,), jnp.float32),
        pltpu.SemaphoreType.DMA,
    ],
)
def _deg_kernel(ei_hbm, out_hbm, dst_v, ones_v, zer_v, accum, ssem):
    cid = lax.axis_index("c")
    sid = lax.axis_index("s")
    wid = _worker_id()
    base = _chunk_base(wid)

    def fill_ones(i, _):
        ones_v[pl.ds(i * 16, 16)] = _z16() + 1.0
        return 0

    lax.fori_loop(0, CK // 16, fill_ones, 0)

    def fill_zero(i, _):
        zer_v[pl.ds(i * 16, 16)] = _z16()
        return 0

    lax.fori_loop(0, RPT // 16, fill_zero, 0)

    pltpu.sync_copy(zer_v, accum.at[pl.ds(sid * RPT, RPT)])
    pltpu.sync_copy(ei_hbm.at[1, pl.ds(base, CH)], dst_v.at[pl.ds(0, CH)])

    @pl.when(wid < XW)
    def _():
        pltpu.sync_copy(ei_hbm.at[1, base + CH], dst_v.at[CH])

    plsc.subcore_barrier()

    def group(g, _):
        gb = g * NBUF
        sd = [
            pltpu.async_copy(
                ones_v, accum.at[dst_v.at[gb + b]], ssem, add=True
            )
            for b in range(NBUF)
        ]
        for d in sd:
            d.wait()
        return 0

    lax.fori_loop(0, CH // NBUF, group, 0)

    @pl.when(wid < XW)
    def _():
        pltpu.sync_copy(ones_v, accum.at[dst_v.at[CH]], add=True)

    plsc.subcore_barrier()
    pltpu.sync_copy(
        accum.at[pl.ds(sid * RPT, RPT)],
        out_hbm.at[cid, pl.ds(sid * RPT, RPT)],
    )


BK = 3
NGR = CH // BK


@functools.partial(
    pl.kernel,
    out_type=jax.ShapeDtypeStruct((2, NP, H), jnp.float32),
    mesh=_mesh,
    compiler_params=_sc_params,
    scratch_types=[
        pltpu.VMEM((CH + 1, CK), jnp.int32),
        pltpu.VMEM((CH + 1, CK), jnp.int32),
        pltpu.VMEM((2 * BK, CK, H), jnp.float32),
        pltpu.VMEM_SHARED((NP, H), jnp.float32),
        pltpu.SemaphoreType.DMA,
        pltpu.SemaphoreType.DMA,
    ],
)
def _agg_feat_kernel(tbl_hbm, ei_hbm, out_hbm, src_v, dst_v, rows_v,
                     accum, gsem, ssem):
    cid = lax.axis_index("c")
    sid = lax.axis_index("s")
    wid = _worker_id()
    base = _chunk_base(wid)

    def fill_zero(i, _):
        rows_v[0, i, pl.ds(0, 16)] = _z16()
        rows_v[0, i, pl.ds(16, 16)] = _z16()
        return 0

    lax.fori_loop(0, CK, fill_zero, 0)

    def zcopy(i, _):
        pltpu.sync_copy(
            rows_v.at[0], accum.at[pl.ds(sid * RPT + i * CK, CK)]
        )
        return 0

    lax.fori_loop(0, RPT // CK, zcopy, 0)
    pltpu.sync_copy(ei_hbm.at[0, pl.ds(base, CH)], src_v.at[pl.ds(0, CH)])
    pltpu.sync_copy(ei_hbm.at[1, pl.ds(base, CH)], dst_v.at[pl.ds(0, CH)])

    @pl.when(wid < XW)
    def _():
        pltpu.sync_copy(ei_hbm.at[0, base + CH], src_v.at[CH])
        pltpu.sync_copy(ei_hbm.at[1, base + CH], dst_v.at[CH])

    plsc.subcore_barrier()

    def gath(j, b):
        return pltpu.async_copy(
            tbl_hbm.at[src_v.at[j]], rows_v.at[b], gsem
        )

    def gwait(j, b):
        pltpu.make_async_copy(
            tbl_hbm.at[src_v.at[j]], rows_v.at[b], gsem
        ).wait()

    def scat(j, b):
        return pltpu.async_copy(
            rows_v.at[b], accum.at[dst_v.at[j]], ssem, add=True
        )

    for k in range(BK):
        gath(k, k)

    def group(g, _):
        bb = (g % 2) * BK
        nb = BK - bb
        for k in range(BK):
            gwait(g * BK + k, bb + k)

        @pl.when(g < NGR - 1)
        def _():
            for k in range(BK):
                gath((g + 1) * BK + k, nb + k)

        sd = [scat(g * BK + k, bb + k) for k in range(BK)]
        for d in sd:
            d.wait()
        return 0

    lax.fori_loop(0, NGR, group, 0)

    @pl.when(wid < XW)
    def _():
        gath(CH, 0).wait()
        scat(CH, 0).wait()

    plsc.subcore_barrier()
    pltpu.sync_copy(
        accum.at[pl.ds(sid * RPT, RPT)],
        out_hbm.at[cid, pl.ds(sid * RPT, RPT)],
    )


@functools.partial(
    pl.kernel,
    out_type=jax.ShapeDtypeStruct((2, NP), jnp.float32),
    mesh=_mesh,
    compiler_params=_sc_params_nl,
    scratch_types=[
        pltpu.VMEM((CH + 1, CK), jnp.int32),
        pltpu.VMEM((CH + 1, CK), jnp.int32),
        pltpu.VMEM((NP,), jnp.float32),
        pltpu.VMEM((NP,), jnp.float32),
        pltpu.VMEM((16, RPT), jnp.float32),
        pltpu.VMEM_SHARED((16, NP), jnp.float32),
    ],
)
def _agg_scal_kernel(tbl_hbm, ei_hbm, out_hbm, src_v, dst_v, z_v, acc_v,
                     red_v, stage):
    cid = lax.axis_index("c")
    sid = lax.axis_index("s")
    wid = _worker_id()
    base = _chunk_base(wid)

    pltpu.sync_copy(tbl_hbm, z_v)
    pltpu.sync_copy(ei_hbm.at[0, pl.ds(base, CH)], src_v.at[pl.ds(0, CH)])
    pltpu.sync_copy(ei_hbm.at[1, pl.ds(base, CH)], dst_v.at[pl.ds(0, CH)])

    @pl.when(wid < XW)
    def _():
        pltpu.sync_copy(ei_hbm.at[0, base + CH], src_v.at[CH])
        pltpu.sync_copy(ei_hbm.at[1, base + CH], dst_v.at[CH])

    def zero(i, _):
        acc_v[pl.ds(i * 16, 16)] = _z16()
        return 0

    lax.fori_loop(0, NP // 16, zero, 0)

    def chunk(i, _):
        def sub(j, _):
            s_idx = src_v[i, pl.ds(j * 16, 16)]
            d_idx = dst_v[i, pl.ds(j * 16, 16)]
            vals = plsc.load_gather(z_v, [s_idx])
            plsc.addupdate_scatter(acc_v, [d_idx], vals)
            return 0

        lax.fori_loop(0, CK // 16, sub, 0)
        return 0

    lax.fori_loop(0, CH, chunk, 0)

    @pl.when(wid < XW)
    def _():
        def sub(j, _):
            s_idx = src_v[CH, pl.ds(j * 16, 16)]
            d_idx = dst_v[CH, pl.ds(j * 16, 16)]
            vals = plsc.load_gather(z_v, [s_idx])
            plsc.addupdate_scatter(acc_v, [d_idx], vals)
            return 0

        lax.fori_loop(0, CK // 16, sub, 0)

    pltpu.sync_copy(acc_v, stage.at[sid])
    plsc.subcore_barrier()
    for t in range(16):
        pltpu.sync_copy(stage.at[t, pl.ds(sid * RPT, RPT)], red_v.at[t])

    def redsum(r, _):
        s = red_v[0, pl.ds(r * 16, 16)]
        for t in range(1, 16):
            s = s + red_v[t, pl.ds(r * 16, 16)]
        acc_v[pl.ds(r * 16, 16)] = s
        return 0

    lax.fori_loop(0, RPT // 16, redsum, 0)
    pltpu.sync_copy(
        acc_v.at[pl.ds(0, RPT)],
        out_hbm.at[cid, pl.ds(sid * RPT, RPT)],
    )


BN = 1024


def _k1_body(x_ref, w1t_ref, degp_ref, h1_ref, h1p_ref, dinv_ref):
    h1 = lax.dot_general(
        w1t_ref[...], x_ref[...], (((1,), (1,)), ((), ())),
        preferred_element_type=jnp.float32,
    )
    degp = degp_ref[...]
    deg = degp[0:1, :] + degp[1:2, :] + 1.0
    dinv = lax.rsqrt(deg)
    h1_ref[...] = h1
    h1p_ref[...] = h1 * dinv
    dinv_ref[...] = dinv


def _k1(xp, w1t, degp):
    return pl.pallas_call(
        _k1_body,
        grid=(NP // BN,),
        in_specs=[
            pl.BlockSpec((BN, D), lambda i: (i, 0)),
            pl.BlockSpec((H, D), lambda i: (0, 0)),
            pl.BlockSpec((2, BN), lambda i: (0, i)),
        ],
        out_specs=[
            pl.BlockSpec((H, BN), lambda i: (0, i)),
            pl.BlockSpec((H, BN), lambda i: (0, i)),
            pl.BlockSpec((1, BN), lambda i: (0, i)),
        ],
        out_shape=[
            jax.ShapeDtypeStruct((H, NP), jnp.float32),
            jax.ShapeDtypeStruct((H, NP), jnp.float32),
            jax.ShapeDtypeStruct((1, NP), jnp.float32),
        ],
    )(xp, w1t, degp)


def _k2_body(gt_ref, h1_ref, dinv_ref, b1c_ref, w2r_ref, h_ref, z_ref):
    g = gt_ref[0] + gt_ref[1]
    dinv = dinv_ref[...]
    acc = dinv * (g + dinv * h1_ref[...]) + b1c_ref[...]
    h = jnp.maximum(acc, 0.0)
    h_ref[...] = h
    h2 = lax.dot_general(
        w2r_ref[...], h, (((1,), (0,)), ((), ())),
        preferred_element_type=jnp.float32,
    )
    z_ref[...] = dinv * h2


def _k2(gt, h1, dinv, b1c, w2r):
    return pl.pallas_call(
        _k2_body,
        grid=(NP // BN,),
        in_specs=[
            pl.BlockSpec((2, H, BN), lambda i: (0, 0, i)),
            pl.BlockSpec((H, BN), lambda i: (0, i)),
            pl.BlockSpec((1, BN), lambda i: (0, i)),
            pl.BlockSpec((H, 1), lambda i: (0, 0)),
            pl.BlockSpec((1, H), lambda i: (0, 0)),
        ],
        out_specs=[
            pl.BlockSpec((H, BN), lambda i: (0, i)),
            pl.BlockSpec((1, BN), lambda i: (0, i)),
        ],
        out_shape=[
            jax.ShapeDtypeStruct((H, NP), jnp.float32),
            jax.ShapeDtypeStruct((1, NP), jnp.float32),
        ],
    )(gt, h1, dinv, b1c, w2r)


def _k3_body(g2p_ref, z_ref, dinv_ref, b2_ref, out_ref):
    g2p = g2p_ref[...]
    g2 = g2p[0:1, :] + g2p[1:2, :]
    out_ref[...] = dinv_ref[...] * (g2 + z_ref[...]) + b2_ref[0]


def _k3(g2p, z, dinv, b2):
    return pl.pallas_call(
        _k3_body,
        in_specs=[
            pl.BlockSpec((2, NP), lambda: (0, 0)),
            pl.BlockSpec((1, NP), lambda: (0, 0)),
            pl.BlockSpec((1, NP), lambda: (0, 0)),
            pl.BlockSpec(memory_space=pltpu.SMEM),
        ],
        out_specs=pl.BlockSpec((1, NP), lambda: (0, 0)),
        out_shape=jax.ShapeDtypeStruct((1, NP), jnp.float32),
    )(g2p, z, dinv, b2)


@jax.jit
def kernel(x, edge_index, W1, b1, W2, b2):
    eir = edge_index.astype(jnp.int32).reshape(2, EC, CK)

    degp = _deg_kernel(eir)
    xp = jnp.pad(x, ((0, NP - N), (0, 0)))
    h1T, h1pT, dinv = _k1(xp, W1.T, degp)

    gp = _agg_feat_kernel(h1pT.T, eir)
    gt = jnp.transpose(gp, (0, 2, 1))
    hT, z = _k2(gt, h1T, dinv, b1.reshape(H, 1), W2.reshape(1, H))

    g2p = _agg_scal_kernel(z.reshape(NP), eir)
    scores = _k3(g2p, z, dinv, b2)

    return hT.T[:N], scores[0, :N]


# speedup vs baseline: 79.7232x; 1.0410x over previous
import functools
import jax
import jax.numpy as jnp
from jax import lax
from jax.experimental import pallas as pl
from jax.experimental.pallas import tpu as pltpu
from jax.experimental.pallas import tpu_sc as plsc

N = 10000
D = 128
H = 32
E = 320000
NP = 10240
NW = 32
CK = 128
EC = E // CK
CH = 78
XW = EC - NW * CH
RPT = NP // 16
NBUF = 6

_mesh = plsc.VectorSubcoreMesh(core_axis_name="c", subcore_axis_name="s")
_sc_params = pltpu.CompilerParams(use_tc_tiling_on_sc=False)
_sc_params_nl = pltpu.CompilerParams(
    use_tc_tiling_on_sc=False, needs_layout_passes=False
)


def _z16():
    return jnp.zeros((16,), jnp.float32)


def _worker_id():
    return lax.axis_index("s") * 2 + lax.axis_index("c")


def _chunk_base(wid):
    return jnp.where(wid < XW, (CH + 1) * wid, XW + CH * wid)


@functools.partial(
    pl.kernel,
    out_type=jax.ShapeDtypeStruct((2, NP), jnp.float32),
    mesh=_mesh,
    compiler_params=_sc_params,
    scratch_types=[
        pltpu.VMEM((CH + 1, CK), jnp.int32),
        pltpu.VMEM((CK,), jnp.float32),
        pltpu.VMEM((RPT,), jnp.float32),
        pltpu.VMEM_SHARED((NP,), jnp.float32),
        pltpu.SemaphoreType.DMA,
    ],
)
def _deg_kernel(ei_hbm, out_hbm, dst_v, ones_v, zer_v, accum, ssem):
    cid = lax.axis_index("c")
    sid = lax.axis_index("s")
    wid = _worker_id()
    base = _chunk_base(wid)

    def fill_ones(i, _):
        ones_v[pl.ds(i * 16, 16)] = _z16() + 1.0
        return 0

    lax.fori_loop(0, CK // 16, fill_ones, 0)

    def fill_zero(i, _):
        zer_v[pl.ds(i * 16, 16)] = _z16()
        return 0

    lax.fori_loop(0, RPT // 16, fill_zero, 0)

    pltpu.sync_copy(zer_v, accum.at[pl.ds(sid * RPT, RPT)])
    pltpu.sync_copy(ei_hbm.at[1, pl.ds(base, CH)], dst_v.at[pl.ds(0, CH)])

    @pl.when(wid < XW)
    def _():
        pltpu.sync_copy(ei_hbm.at[1, base + CH], dst_v.at[CH])

    plsc.subcore_barrier()

    def group(g, _):
        gb = g * NBUF
        sd = [
            pltpu.async_copy(
                ones_v, accum.at[dst_v.at[gb + b]], ssem, add=True
            )
            for b in range(NBUF)
        ]
        for d in sd:
            d.wait()
        return 0

    lax.fori_loop(0, CH // NBUF, group, 0)

    @pl.when(wid < XW)
    def _():
        pltpu.sync_copy(ones_v, accum.at[dst_v.at[CH]], add=True)

    plsc.subcore_barrier()
    pltpu.sync_copy(
        accum.at[pl.ds(sid * RPT, RPT)],
        out_hbm.at[cid, pl.ds(sid * RPT, RPT)],
    )


BK = 3
NGR = CH // BK


@functools.partial(
    pl.kernel,
    out_type=jax.ShapeDtypeStruct((2, NP, H), jnp.float32),
    mesh=_mesh,
    compiler_params=_sc_params,
    scratch_types=[
        pltpu.VMEM((CH + 1, CK), jnp.int32),
        pltpu.VMEM((CH + 1, CK), jnp.int32),
        pltpu.VMEM((2 * BK, CK, H), jnp.float32),
        pltpu.VMEM_SHARED((NP, H), jnp.float32),
        pltpu.SemaphoreType.DMA,
        pltpu.SemaphoreType.DMA,
    ],
)
def _agg_feat_kernel(tbl_hbm, ei_hbm, out_hbm, src_v, dst_v, rows_v,
                     accum, gsem, ssem):
    cid = lax.axis_index("c")
    sid = lax.axis_index("s")
    wid = _worker_id()
    base = _chunk_base(wid)

    def fill_zero(i, _):
        rows_v[0, i, pl.ds(0, 16)] = _z16()
        rows_v[0, i, pl.ds(16, 16)] = _z16()
        return 0

    lax.fori_loop(0, CK, fill_zero, 0)

    def zcopy(i, _):
        pltpu.sync_copy(
            rows_v.at[0], accum.at[pl.ds(sid * RPT + i * CK, CK)]
        )
        return 0

    lax.fori_loop(0, RPT // CK, zcopy, 0)
    pltpu.sync_copy(ei_hbm.at[0, pl.ds(base, CH)], src_v.at[pl.ds(0, CH)])
    pltpu.sync_copy(ei_hbm.at[1, pl.ds(base, CH)], dst_v.at[pl.ds(0, CH)])

    @pl.when(wid < XW)
    def _():
        pltpu.sync_copy(ei_hbm.at[0, base + CH], src_v.at[CH])
        pltpu.sync_copy(ei_hbm.at[1, base + CH], dst_v.at[CH])

    plsc.subcore_barrier()

    def gath(j, b):
        return pltpu.async_copy(
            tbl_hbm.at[src_v.at[j]], rows_v.at[b], gsem
        )

    def gwait(j, b):
        pltpu.make_async_copy(
            tbl_hbm.at[src_v.at[j]], rows_v.at[b], gsem
        ).wait()

    def scat(j, b):
        return pltpu.async_copy(
            rows_v.at[b], accum.at[dst_v.at[j]], ssem, add=True
        )

    for k in range(BK):
        gath(k, k)

    def group(g, _):
        bb = (g % 2) * BK
        nb = BK - bb
        for k in range(BK):
            gwait(g * BK + k, bb + k)

        @pl.when(g < NGR - 1)
        def _():
            for k in range(BK):
                gath((g + 1) * BK + k, nb + k)

        sd = [scat(g * BK + k, bb + k) for k in range(BK)]
        for d in sd:
            d.wait()
        return 0

    lax.fori_loop(0, NGR, group, 0)

    @pl.when(wid < XW)
    def _():
        gath(CH, 0).wait()
        scat(CH, 0).wait()

    plsc.subcore_barrier()
    pltpu.sync_copy(
        accum.at[pl.ds(sid * RPT, RPT)],
        out_hbm.at[cid, pl.ds(sid * RPT, RPT)],
    )


@functools.partial(
    pl.kernel,
    out_type=jax.ShapeDtypeStruct((2, NP), jnp.float32),
    mesh=_mesh,
    compiler_params=_sc_params_nl,
    scratch_types=[
        pltpu.VMEM((CH + 1, CK), jnp.int32),
        pltpu.VMEM((CH + 1, CK), jnp.int32),
        pltpu.VMEM((NP,), jnp.float32),
        pltpu.VMEM((NP,), jnp.float32),
        pltpu.VMEM((16, RPT), jnp.float32),
        pltpu.VMEM_SHARED((16, NP), jnp.float32),
        pltpu.SemaphoreType.DMA,
    ],
)
def _agg_scal_kernel(tbl_hbm, ei_hbm, out_hbm, src_v, dst_v, z_v, acc_v,
                     red_v, stage, lsem):
    cid = lax.axis_index("c")
    sid = lax.axis_index("s")
    wid = _worker_id()
    base = _chunk_base(wid)

    ld = [
        pltpu.async_copy(tbl_hbm, z_v, lsem),
        pltpu.async_copy(
            ei_hbm.at[0, pl.ds(base, CH)], src_v.at[pl.ds(0, CH)], lsem
        ),
        pltpu.async_copy(
            ei_hbm.at[1, pl.ds(base, CH)], dst_v.at[pl.ds(0, CH)], lsem
        ),
    ]

    @pl.when(wid < XW)
    def _():
        pltpu.sync_copy(ei_hbm.at[0, base + CH], src_v.at[CH])
        pltpu.sync_copy(ei_hbm.at[1, base + CH], dst_v.at[CH])

    def zero(i, _):
        acc_v[pl.ds(i * 16, 16)] = _z16()
        return 0

    lax.fori_loop(0, NP // 16, zero, 0)
    for d in ld:
        d.wait()

    def chunk(i, _):
        def sub(j, _):
            s_idx = src_v[i, pl.ds(j * 16, 16)]
            d_idx = dst_v[i, pl.ds(j * 16, 16)]
            vals = plsc.load_gather(z_v, [s_idx])
            plsc.addupdate_scatter(acc_v, [d_idx], vals)
            return 0

        lax.fori_loop(0, CK // 16, sub, 0)
        return 0

    lax.fori_loop(0, CH, chunk, 0)

    @pl.when(wid < XW)
    def _():
        def sub(j, _):
            s_idx = src_v[CH, pl.ds(j * 16, 16)]
            d_idx = dst_v[CH, pl.ds(j * 16, 16)]
            vals = plsc.load_gather(z_v, [s_idx])
            plsc.addupdate_scatter(acc_v, [d_idx], vals)
            return 0

        lax.fori_loop(0, CK // 16, sub, 0)

    pltpu.sync_copy(acc_v, stage.at[sid])
    plsc.subcore_barrier()
    for t in range(16):
        pltpu.sync_copy(stage.at[t, pl.ds(sid * RPT, RPT)], red_v.at[t])

    def redsum(r, _):
        s = red_v[0, pl.ds(r * 16, 16)]
        for t in range(1, 16):
            s = s + red_v[t, pl.ds(r * 16, 16)]
        acc_v[pl.ds(r * 16, 16)] = s
        return 0

    lax.fori_loop(0, RPT // 16, redsum, 0)
    pltpu.sync_copy(
        acc_v.at[pl.ds(0, RPT)],
        out_hbm.at[cid, pl.ds(sid * RPT, RPT)],
    )


BN = 1024


def _k1_body(x_ref, w1t_ref, degp_ref, h1_ref, h1p_ref, dinv_ref):
    h1 = lax.dot_general(
        w1t_ref[...], x_ref[...], (((1,), (1,)), ((), ())),
        preferred_element_type=jnp.float32,
    )
    degp = degp_ref[...]
    deg = degp[0:1, :] + degp[1:2, :] + 1.0
    dinv = lax.rsqrt(deg)
    h1_ref[...] = h1
    h1p_ref[...] = jnp.transpose(h1 * dinv)
    dinv_ref[...] = dinv


def _k1(xp, w1t, degp):
    return pl.pallas_call(
        _k1_body,
        grid=(NP // BN,),
        in_specs=[
            pl.BlockSpec((BN, D), lambda i: (i, 0)),
            pl.BlockSpec((H, D), lambda i: (0, 0)),
            pl.BlockSpec((2, BN), lambda i: (0, i)),
        ],
        out_specs=[
            pl.BlockSpec((H, BN), lambda i: (0, i)),
            pl.BlockSpec((BN, H), lambda i: (i, 0)),
            pl.BlockSpec((1, BN), lambda i: (0, i)),
        ],
        out_shape=[
            jax.ShapeDtypeStruct((H, NP), jnp.float32),
            jax.ShapeDtypeStruct((NP, H), jnp.float32),
            jax.ShapeDtypeStruct((1, NP), jnp.float32),
        ],
    )(xp, w1t, degp)


def _k2_body(gp_ref, h1_ref, dinv_ref, b1c_ref, w2r_ref, h_ref, z_ref):
    g = jnp.transpose(gp_ref[0] + gp_ref[1])
    dinv = dinv_ref[...]
    acc = dinv * (g + dinv * h1_ref[...]) + b1c_ref[...]
    h = jnp.maximum(acc, 0.0)
    h_ref[...] = jnp.transpose(h)
    h2 = lax.dot_general(
        w2r_ref[...], h, (((1,), (0,)), ((), ())),
        preferred_element_type=jnp.float32,
    )
    z_ref[...] = dinv * h2


def _k2(gp, h1, dinv, b1c, w2r):
    return pl.pallas_call(
        _k2_body,
        grid=(NP // BN,),
        in_specs=[
            pl.BlockSpec((2, BN, H), lambda i: (0, i, 0)),
            pl.BlockSpec((H, BN), lambda i: (0, i)),
            pl.BlockSpec((1, BN), lambda i: (0, i)),
            pl.BlockSpec((H, 1), lambda i: (0, 0)),
            pl.BlockSpec((1, H), lambda i: (0, 0)),
        ],
        out_specs=[
            pl.BlockSpec((BN, H), lambda i: (i, 0)),
            pl.BlockSpec((1, BN), lambda i: (0, i)),
        ],
        out_shape=[
            jax.ShapeDtypeStruct((NP, H), jnp.float32),
            jax.ShapeDtypeStruct((1, NP), jnp.float32),
        ],
    )(gp, h1, dinv, b1c, w2r)


def _k3_body(g2p_ref, z_ref, dinv_ref, b2_ref, out_ref):
    g2p = g2p_ref[...]
    g2 = g2p[0:1, :] + g2p[1:2, :]
    out_ref[...] = dinv_ref[...] * (g2 + z_ref[...]) + b2_ref[0]


def _k3(g2p, z, dinv, b2):
    return pl.pallas_call(
        _k3_body,
        in_specs=[
            pl.BlockSpec((2, NP), lambda: (0, 0)),
            pl.BlockSpec((1, NP), lambda: (0, 0)),
            pl.BlockSpec((1, NP), lambda: (0, 0)),
            pl.BlockSpec(memory_space=pltpu.SMEM),
        ],
        out_specs=pl.BlockSpec((1, NP), lambda: (0, 0)),
        out_shape=jax.ShapeDtypeStruct((1, NP), jnp.float32),
    )(g2p, z, dinv, b2)


@jax.jit
def kernel(x, edge_index, W1, b1, W2, b2):
    eir = edge_index.astype(jnp.int32).reshape(2, EC, CK)

    degp = _deg_kernel(eir)
    xp = jnp.pad(x, ((0, NP - N), (0, 0)))
    h1T, h1p, dinv = _k1(xp, W1.T, degp)

    gp = _agg_feat_kernel(h1p, eir)
    h, z = _k2(gp, h1T, dinv, b1.reshape(H, 1), W2.reshape(1, H))

    g2p = _agg_scal_kernel(z.reshape(NP), eir)
    scores = _k3(g2p, z, dinv, b2)

    return h[:N], scores[0, :N]


# speedup vs baseline: 80.7267x; 1.0126x over previous
import functools
import jax
import jax.numpy as jnp
from jax import lax
from jax.experimental import pallas as pl
from jax.experimental.pallas import tpu as pltpu
from jax.experimental.pallas import tpu_sc as plsc

N = 10000
D = 128
H = 32
E = 320000
NP = 10240
NW = 32
CK = 128
EC = E // CK
CH = 78
XW = EC - NW * CH
RPT = NP // 16
NBUF = 6

_mesh = plsc.VectorSubcoreMesh(core_axis_name="c", subcore_axis_name="s")
_sc_params = pltpu.CompilerParams(use_tc_tiling_on_sc=False)
_sc_params_nl = pltpu.CompilerParams(
    use_tc_tiling_on_sc=False, needs_layout_passes=False
)


def _z16():
    return jnp.zeros((16,), jnp.float32)


def _worker_id():
    return lax.axis_index("s") * 2 + lax.axis_index("c")


def _chunk_base(wid):
    return jnp.where(wid < XW, (CH + 1) * wid, XW + CH * wid)


@functools.partial(
    pl.kernel,
    out_type=jax.ShapeDtypeStruct((2, NP), jnp.float32),
    mesh=_mesh,
    compiler_params=_sc_params,
    scratch_types=[
        pltpu.VMEM((CH + 1, CK), jnp.int32),
        pltpu.VMEM((CK,), jnp.float32),
        pltpu.VMEM((RPT,), jnp.float32),
        pltpu.VMEM_SHARED((NP,), jnp.float32),
        pltpu.SemaphoreType.DMA,
    ],
)
def _deg_kernel(ei_hbm, out_hbm, dst_v, ones_v, zer_v, accum, ssem):
    cid = lax.axis_index("c")
    sid = lax.axis_index("s")
    wid = _worker_id()
    base = _chunk_base(wid)

    def fill_ones(i, _):
        ones_v[pl.ds(i * 16, 16)] = _z16() + 1.0
        return 0

    lax.fori_loop(0, CK // 16, fill_ones, 0)

    def fill_zero(i, _):
        zer_v[pl.ds(i * 16, 16)] = _z16()
        return 0

    lax.fori_loop(0, RPT // 16, fill_zero, 0)

    pltpu.sync_copy(zer_v, accum.at[pl.ds(sid * RPT, RPT)])
    pltpu.sync_copy(ei_hbm.at[1, pl.ds(base, CH)], dst_v.at[pl.ds(0, CH)])

    @pl.when(wid < XW)
    def _():
        pltpu.sync_copy(ei_hbm.at[1, base + CH], dst_v.at[CH])

    plsc.subcore_barrier()

    def group(g, _):
        gb = g * NBUF
        sd = [
            pltpu.async_copy(
                ones_v, accum.at[dst_v.at[gb + b]], ssem, add=True
            )
            for b in range(NBUF)
        ]
        for d in sd:
            d.wait()
        return 0

    lax.fori_loop(0, CH // NBUF, group, 0)

    @pl.when(wid < XW)
    def _():
        pltpu.sync_copy(ones_v, accum.at[dst_v.at[CH]], add=True)

    plsc.subcore_barrier()
    pltpu.sync_copy(
        accum.at[pl.ds(sid * RPT, RPT)],
        out_hbm.at[cid, pl.ds(sid * RPT, RPT)],
    )


BK = 3
NGR = CH // BK


@functools.partial(
    pl.kernel,
    out_type=jax.ShapeDtypeStruct((2, NP, H), jnp.float32),
    mesh=_mesh,
    compiler_params=_sc_params,
    scratch_types=[
        pltpu.VMEM((CH + 1, CK), jnp.int32),
        pltpu.VMEM((CH + 1, CK), jnp.int32),
        pltpu.VMEM((2 * BK, CK, H), jnp.float32),
        pltpu.VMEM_SHARED((NP, H), jnp.float32),
        pltpu.SemaphoreType.DMA,
        pltpu.SemaphoreType.DMA,
    ],
)
def _agg_feat_kernel(tbl_hbm, ei_hbm, out_hbm, src_v, dst_v, rows_v,
                     accum, gsem, ssem):
    cid = lax.axis_index("c")
    sid = lax.axis_index("s")
    wid = _worker_id()
    base = _chunk_base(wid)

    def fill_zero(i, _):
        rows_v[0, i, pl.ds(0, 16)] = _z16()
        rows_v[0, i, pl.ds(16, 16)] = _z16()
        return 0

    lax.fori_loop(0, CK, fill_zero, 0)

    def zcopy(i, _):
        pltpu.sync_copy(
            rows_v.at[0], accum.at[pl.ds(sid * RPT + i * CK, CK)]
        )
        return 0

    lax.fori_loop(0, RPT // CK, zcopy, 0)
    pltpu.sync_copy(ei_hbm.at[0, pl.ds(base, CH)], src_v.at[pl.ds(0, CH)])
    pltpu.sync_copy(ei_hbm.at[1, pl.ds(base, CH)], dst_v.at[pl.ds(0, CH)])

    @pl.when(wid < XW)
    def _():
        pltpu.sync_copy(ei_hbm.at[0, base + CH], src_v.at[CH])
        pltpu.sync_copy(ei_hbm.at[1, base + CH], dst_v.at[CH])

    plsc.subcore_barrier()

    def gath(j, b):
        return pltpu.async_copy(
            tbl_hbm.at[src_v.at[j]], rows_v.at[b], gsem
        )

    def gwait(j, b):
        pltpu.make_async_copy(
            tbl_hbm.at[src_v.at[j]], rows_v.at[b], gsem
        ).wait()

    def scat(j, b):
        return pltpu.async_copy(
            rows_v.at[b], accum.at[dst_v.at[j]], ssem, add=True
        )

    for k in range(BK):
        gath(k, k)

    def group(g, _):
        bb = (g % 2) * BK
        nb = BK - bb
        for k in range(BK):
            gwait(g * BK + k, bb + k)

        @pl.when(g < NGR - 1)
        def _():
            for k in range(BK):
                gath((g + 1) * BK + k, nb + k)

        sd = [scat(g * BK + k, bb + k) for k in range(BK)]
        for d in sd:
            d.wait()
        return 0

    lax.fori_loop(0, NGR, group, 0)

    @pl.when(wid < XW)
    def _():
        gath(CH, 0).wait()
        scat(CH, 0).wait()

    plsc.subcore_barrier()
    pltpu.sync_copy(
        accum.at[pl.ds(sid * RPT, RPT)],
        out_hbm.at[cid, pl.ds(sid * RPT, RPT)],
    )


@functools.partial(
    pl.kernel,
    out_type=jax.ShapeDtypeStruct((2, NP), jnp.float32),
    mesh=_mesh,
    compiler_params=_sc_params_nl,
    scratch_types=[
        pltpu.VMEM((CH + 1, CK), jnp.int32),
        pltpu.VMEM((CH + 1, CK), jnp.int32),
        pltpu.VMEM((NP,), jnp.float32),
        pltpu.VMEM((NP,), jnp.float32),
        pltpu.VMEM((16, RPT), jnp.float32),
        pltpu.VMEM_SHARED((16, NP), jnp.float32),
        pltpu.SemaphoreType.DMA,
    ],
)
def _agg_scal_kernel(tbl_hbm, ei_hbm, out_hbm, src_v, dst_v, z_v, acc_v,
                     red_v, stage, lsem):
    cid = lax.axis_index("c")
    sid = lax.axis_index("s")
    wid = _worker_id()
    base = _chunk_base(wid)

    ld = [
        pltpu.async_copy(tbl_hbm, z_v, lsem),
        pltpu.async_copy(
            ei_hbm.at[0, pl.ds(base, CH)], src_v.at[pl.ds(0, CH)], lsem
        ),
        pltpu.async_copy(
            ei_hbm.at[1, pl.ds(base, CH)], dst_v.at[pl.ds(0, CH)], lsem
        ),
    ]

    @pl.when(wid < XW)
    def _():
        pltpu.sync_copy(ei_hbm.at[0, base + CH], src_v.at[CH])
        pltpu.sync_copy(ei_hbm.at[1, base + CH], dst_v.at[CH])

    def zero(i, _):
        acc_v[pl.ds(i * 16, 16)] = _z16()
        return 0

    lax.fori_loop(0, NP // 16, zero, 0)
    for d in ld:
        d.wait()

    def chunk(i, _):
        def sub(j, _):
            s_idx = src_v[i, pl.ds(j * 16, 16)]
            d_idx = dst_v[i, pl.ds(j * 16, 16)]
            vals = plsc.load_gather(z_v, [s_idx])
            plsc.addupdate_scatter(acc_v, [d_idx], vals)
            return 0

        lax.fori_loop(0, CK // 16, sub, 0)
        return 0

    lax.fori_loop(0, CH, chunk, 0)

    @pl.when(wid < XW)
    def _():
        def sub(j, _):
            s_idx = src_v[CH, pl.ds(j * 16, 16)]
            d_idx = dst_v[CH, pl.ds(j * 16, 16)]
            vals = plsc.load_gather(z_v, [s_idx])
            plsc.addupdate_scatter(acc_v, [d_idx], vals)
            return 0

        lax.fori_loop(0, CK // 16, sub, 0)

    pltpu.sync_copy(acc_v, stage.at[sid])
    plsc.subcore_barrier()
    for t in range(16):
        pltpu.sync_copy(stage.at[t, pl.ds(sid * RPT, RPT)], red_v.at[t])

    def redsum(r, _):
        s = red_v[0, pl.ds(r * 16, 16)]
        for t in range(1, 16):
            s = s + red_v[t, pl.ds(r * 16, 16)]
        acc_v[pl.ds(r * 16, 16)] = s
        return 0

    lax.fori_loop(0, RPT // 16, redsum, 0)
    pltpu.sync_copy(
        acc_v.at[pl.ds(0, RPT)],
        out_hbm.at[cid, pl.ds(sid * RPT, RPT)],
    )


BN = 1024


def _k1_body(x_ref, w1t_ref, degp_ref, h1_ref, h1p_ref, dinv_ref):
    h1 = lax.dot_general(
        w1t_ref[...], x_ref[...], (((1,), (1,)), ((), ())),
        preferred_element_type=jnp.float32,
    )
    degp = degp_ref[...]
    deg = degp[0:1, :] + degp[1:2, :] + 1.0
    dinv = lax.rsqrt(deg)
    h1_ref[...] = h1
    h1p_ref[...] = jnp.transpose(h1 * dinv)
    dinv_ref[...] = dinv


def _k1(xp, w1t, degp):
    return pl.pallas_call(
        _k1_body,
        grid=(NP // BN,),
        in_specs=[
            pl.BlockSpec((BN, D), lambda i: (i, 0)),
            pl.BlockSpec((H, D), lambda i: (0, 0)),
            pl.BlockSpec((2, BN), lambda i: (0, i)),
        ],
        out_specs=[
            pl.BlockSpec((H, BN), lambda i: (0, i)),
            pl.BlockSpec((BN, H), lambda i: (i, 0)),
            pl.BlockSpec((1, BN), lambda i: (0, i)),
        ],
        out_shape=[
            jax.ShapeDtypeStruct((H, NP), jnp.float32),
            jax.ShapeDtypeStruct((NP, H), jnp.float32),
            jax.ShapeDtypeStruct((1, NP), jnp.float32),
        ],
    )(xp, w1t, degp)


def _k2_body(gp_ref, h1_ref, dinv_ref, b1c_ref, w2r_ref, h_ref, z_ref):
    g = jnp.transpose(gp_ref[0] + gp_ref[1])
    dinv = dinv_ref[...]
    acc = dinv * (g + dinv * h1_ref[...]) + b1c_ref[...]
    h = jnp.maximum(acc, 0.0)
    h_ref[...] = jnp.transpose(h)
    h2 = lax.dot_general(
        w2r_ref[...], h, (((1,), (0,)), ((), ())),
        preferred_element_type=jnp.float32,
    )
    z_ref[...] = (dinv * h2).reshape(BN)


def _k2(gp, h1, dinv, b1c, w2r):
    return pl.pallas_call(
        _k2_body,
        grid=(NP // BN,),
        in_specs=[
            pl.BlockSpec((2, BN, H), lambda i: (0, i, 0)),
            pl.BlockSpec((H, BN), lambda i: (0, i)),
            pl.BlockSpec((1, BN), lambda i: (0, i)),
            pl.BlockSpec((H, 1), lambda i: (0, 0)),
            pl.BlockSpec((1, H), lambda i: (0, 0)),
        ],
        out_specs=[
            pl.BlockSpec((BN, H), lambda i: (i, 0)),
            pl.BlockSpec((BN,), lambda i: (i,)),
        ],
        out_shape=[
            jax.ShapeDtypeStruct((N, H), jnp.float32),
            jax.ShapeDtypeStruct((NP,), jnp.float32),
        ],
    )(gp, h1, dinv, b1c, w2r)


def _k3_body(g2p_ref, z_ref, dinv_ref, b2_ref, out_ref):
    g2p = g2p_ref[...]
    g2 = g2p[0:1, :] + g2p[1:2, :]
    z = z_ref[...].reshape(1, NP)
    out_ref[...] = dinv_ref[...] * (g2 + z) + b2_ref[0]


def _k3(g2p, z, dinv, b2):
    return pl.pallas_call(
        _k3_body,
        in_specs=[
            pl.BlockSpec((2, NP), lambda: (0, 0)),
            pl.BlockSpec((NP,), lambda: (0,)),
            pl.BlockSpec((1, NP), lambda: (0, 0)),
            pl.BlockSpec(memory_space=pltpu.SMEM),
        ],
        out_specs=pl.BlockSpec((1, NP), lambda: (0, 0)),
        out_shape=jax.ShapeDtypeStruct((1, NP), jnp.float32),
    )(g2p, z, dinv, b2)


@jax.jit
def kernel(x, edge_index, W1, b1, W2, b2):
    eir = edge_index.astype(jnp.int32).reshape(2, EC, CK)

    degp = _deg_kernel(eir)
    xp = jnp.pad(x, ((0, NP - N), (0, 0)))
    h1T, h1p, dinv = _k1(xp, W1.T, degp)

    gp = _agg_feat_kernel(h1p, eir)
    h, z = _k2(gp, h1T, dinv, b1.reshape(H, 1), W2.reshape(1, H))

    g2p = _agg_scal_kernel(z, eir)
    scores = _k3(g2p, z, dinv, b2)

    return h, scores[0, :N]
